# Initial kernel scaffold; baseline (speedup 1.0000x reference)
import jax
import jax.numpy as jnp
from jax.experimental import pallas as pl


def kernel(x, edge_index, edge_attr, batch, x_clique_5_cycle, atom2clique_5_cycle, x_clique_6_cycle, atom2clique_6_cycle, params):
    raise NotImplementedError("write your pallas kernel here")


# speedup vs baseline: 3.0243x; 3.0243x over previous
import functools

import jax
import jax.numpy as jnp
from jax import lax
from jax.experimental import pallas as pl
from jax.experimental.pallas import tpu as pltpu
from jax.experimental.pallas import tpu_sc as plsc

H = 128
N = 10000
NP = 10112
NG = 512
E = 320000
KC = 128
KS = 80
NWORK = 32
ECH = 80
EPAD = NWORK * ECH * KC
CSZ = {'5': 2000, '6': 2500}
CPAD = {'5': 2048, '6': 2560}
PCH = {'5': 3, '6': 4}
PPAD = {'5': NWORK * 3 * KC, '6': NWORK * 4 * KC}
ZR = 128
F32 = jnp.float32

_MESH = plsc.VectorSubcoreMesh(core_axis_name="c", subcore_axis_name="s")


def _fill_const(zbuf, rows, vec16s):
    def body(i, _):
        for j in range(H // 16):
            zbuf[i, pl.ds(16 * j, 16)] = vec16s[j]
        return 0
    lax.fori_loop(0, rows, body, 0)


def _zero_stripe(acc, zbuf, stripe, sid):
    base = sid * stripe
    nfull = stripe // ZR
    rem = stripe % ZR
    for q in range(nfull):
        pltpu.sync_copy(zbuf, acc.at[pl.ds(base + q * ZR, ZR)])
    if rem:
        pltpu.sync_copy(zbuf.at[pl.ds(0, rem)],
                        acc.at[pl.ds(base + nfull * ZR, rem)])


def _zeros16():
    return [jnp.zeros((16,), F32)] * (H // 16)


def _edge_sc(h, tbl, srcp, dstp, codep):
    @functools.partial(
        pl.kernel, mesh=_MESH,
        out_type=jax.ShapeDtypeStruct((2, NP, H), F32),
        scratch_types=[
            pltpu.VMEM((KC,), jnp.int32),
            pltpu.VMEM((KC,), jnp.int32),
            pltpu.VMEM((KC,), jnp.int32),
            pltpu.VMEM((KC, H), F32),
            pltpu.VMEM((KC, H), F32),
            pltpu.VMEM((ZR, H), F32),
            pltpu.VMEM_SHARED((NP, H), F32),
            pltpu.SemaphoreType.DMA,
            pltpu.SemaphoreType.DMA,
        ],
    )
    def k(h_hbm, t_hbm, src_hbm, dst_hbm, code_hbm, out_hbm,
          sbuf, dbuf, cbuf, hrows, trows, zbuf, acc, semh, semt):
        c = lax.axis_index("c")
        s = lax.axis_index("s")
        w = c * 16 + s
        _fill_const(zbuf, ZR, _zeros16())
        stripe = NP // 16
        _zero_stripe(acc, zbuf, stripe, s)
        plsc.subcore_barrier()

        def chunk(g, _):
            base = w * (ECH * KC) + g * KC
            pltpu.sync_copy(src_hbm.at[pl.ds(base, KC)], sbuf)
            pltpu.sync_copy(dst_hbm.at[pl.ds(base, KC)], dbuf)
            pltpu.sync_copy(code_hbm.at[pl.ds(base, KC)], cbuf)
            cph = pltpu.async_copy(h_hbm.at[sbuf], hrows, semh)
            cpt = pltpu.async_copy(t_hbm.at[cbuf], trows, semt)
            cph.wait()
            cpt.wait()

            def vrow(i, _):
                for j in range(H // 16):
                    sl = pl.ds(16 * j, 16)
                    trows[i, sl] = jnp.maximum(hrows[i, sl] + trows[i, sl],
                                               0.0)
                return 0
            lax.fori_loop(0, KC, vrow, 0)
            pltpu.sync_copy(trows, acc.at[dbuf], add=True)
            return 0
        lax.fori_loop(0, ECH, chunk, 0)
        plsc.subcore_barrier()
        pltpu.sync_copy(acc.at[pl.ds(s * stripe, stripe)],
                        out_hbm.at[c, pl.ds(s * stripe, stripe)])

    return k(h, tbl, srcp, dstp, codep)


def _gs_sc(tab, gidx, sidx, acc_rows, nchunks):
    @functools.partial(
        pl.kernel, mesh=_MESH,
        out_type=jax.ShapeDtypeStruct((2, acc_rows, H), F32),
        scratch_types=[
            pltpu.VMEM((KC,), jnp.int32),
            pltpu.VMEM((KC,), jnp.int32),
            pltpu.VMEM((KC, H), F32),
            pltpu.VMEM((ZR, H), F32),
            pltpu.VMEM_SHARED((acc_rows, H), F32),
            pltpu.SemaphoreType.DMA,
        ],
    )
    def k(tab_hbm, gidx_hbm, sidx_hbm, out_hbm, gbuf, sbuf, rows, zbuf, acc,
          sem):
        c = lax.axis_index("c")
        s = lax.axis_index("s")
        w = c * 16 + s
        _fill_const(zbuf, ZR, _zeros16())
        stripe = acc_rows // 16
        _zero_stripe(acc, zbuf, stripe, s)
        plsc.subcore_barrier()

        def chunk(g, _):
            base = w * (nchunks * KC) + g * KC
            pltpu.sync_copy(gidx_hbm.at[pl.ds(base, KC)], gbuf)
            pltpu.sync_copy(sidx_hbm.at[pl.ds(base, KC)], sbuf)
            pltpu.async_copy(tab_hbm.at[gbuf], rows, sem).wait()
            pltpu.sync_copy(rows, acc.at[sbuf], add=True)
            return 0
        lax.fori_loop(0, nchunks, chunk, 0)
        plsc.subcore_barrier()
        pltpu.sync_copy(acc.at[pl.ds(s * stripe, stripe)],
                        out_hbm.at[c, pl.ds(s * stripe, stripe)])

    return k(tab, gidx, sidx)


def _hist_sc(sidx, acc_rows, kchunk, nchunks_total):
    nloop = -(-nchunks_total // NWORK)

    @functools.partial(
        pl.kernel, mesh=_MESH,
        out_type=jax.ShapeDtypeStruct((2, acc_rows, H), F32),
        scratch_types=[
            pltpu.VMEM((kchunk,), jnp.int32),
            pltpu.VMEM((kchunk, H), F32),
            pltpu.VMEM((ZR, H), F32),
            pltpu.VMEM_SHARED((acc_rows, H), F32),
        ],
    )
    def k(sidx_hbm, out_hbm, sbuf, rows, zbuf, acc):
        c = lax.axis_index("c")
        s = lax.axis_index("s")
        w = c * 16 + s
        _fill_const(zbuf, ZR, _zeros16())
        stripe = acc_rows // 16
        _zero_stripe(acc, zbuf, stripe, s)
        lane = lax.iota(jnp.int32, 16)
        one16 = jnp.where(lane == 0, jnp.float32(1.0), jnp.float32(0.0))
        _fill_const(rows, kchunk, [one16] + _zeros16()[1:])
        plsc.subcore_barrier()
        for t in range(nloop):
            j = w + NWORK * t

            @pl.when(j < nchunks_total)
            def _():
                base = j * kchunk
                pltpu.sync_copy(sidx_hbm.at[pl.ds(base, kchunk)], sbuf)
                pltpu.sync_copy(rows, acc.at[sbuf], add=True)
        plsc.subcore_barrier()
        pltpu.sync_copy(acc.at[pl.ds(s * stripe, stripe)],
                        out_hbm.at[c, pl.ds(s * stripe, stripe)])

    return k(sidx)


def _embed_sc(tabA, xind):
    @functools.partial(
        pl.kernel, mesh=_MESH,
        out_type=jax.ShapeDtypeStruct((N, H), F32),
        scratch_types=[
            pltpu.VMEM((KS,), jnp.int32),
            pltpu.VMEM((KS, H), F32),
            pltpu.VMEM((KS, H), F32),
            pltpu.SemaphoreType.DMA,
        ],
    )
    def k(tab_hbm, xind_hbm, out_hbm, idxb, rows, accb, sem):
        c = lax.axis_index("c")
        s = lax.axis_index("s")
        w = c * 16 + s
        for t in range(4):
            j = w + NWORK * t

            @pl.when(j < N // KS)
            def _():
                base = j * KS
                _fill_const(accb, KS, _zeros16())
                for f in range(9):
                    pltpu.sync_copy(xind_hbm.at[pl.ds(f * N + base, KS)],
                                    idxb)
                    pltpu.async_copy(tab_hbm.at[idxb], rows, sem).wait()

                    def addrow(i, _):
                        for jj in range(H // 16):
                            sl = pl.ds(16 * jj, 16)
                            accb[i, sl] = accb[i, sl] + rows[i, sl]
                        return 0
                    lax.fori_loop(0, KS, addrow, 0)
                pltpu.sync_copy(accb, out_hbm.at[pl.ds(base, KS)])

    return k(tabA, xind)


def _pool_sc(h, batch):
    @functools.partial(
        pl.kernel, mesh=_MESH,
        out_type=jax.ShapeDtypeStruct((2, NG, H), F32),
        scratch_types=[
            pltpu.VMEM((KS,), jnp.int32),
            pltpu.VMEM((KS, H), F32),
            pltpu.VMEM((ZR, H), F32),
            pltpu.VMEM_SHARED((NG, H), F32),
        ],
    )
    def k(h_hbm, b_hbm, out_hbm, idxb, rows, zbuf, acc):
        c = lax.axis_index("c")
        s = lax.axis_index("s")
        w = c * 16 + s
        _fill_const(zbuf, ZR, _zeros16())
        stripe = NG // 16
        _zero_stripe(acc, zbuf, stripe, s)
        plsc.subcore_barrier()
        for t in range(4):
            j = w + NWORK * t

            @pl.when(j < N // KS)
            def _():
                base = j * KS
                pltpu.sync_copy(b_hbm.at[pl.ds(base, KS)], idxb)
                pltpu.sync_copy(h_hbm.at[pl.ds(base, KS)], rows)
                pltpu.sync_copy(rows, acc.at[idxb], add=True)
        plsc.subcore_barrier()
        pltpu.sync_copy(acc.at[pl.ds(s * stripe, stripe)],
                        out_hbm.at[c, pl.ds(s * stripe, stripe)])

    return k(h, batch)


def _cnt_tc(hist, rows):
    def body(h_ref, o_ref):
        sm = h_ref[0][:, 0:1] + h_ref[1][:, 0:1]
        o_ref[...] = jnp.broadcast_to(sm, (rows, 8))
    return pl.pallas_call(
        body,
        out_shape=jax.ShapeDtypeStruct((rows, 8), F32),
    )(hist)


def _tbuild(bond_emb):
    def body(be_ref, t_ref):
        b0 = be_ref[0]
        b1 = be_ref[1]
        b2 = be_ref[2]
        for a in range(8):
            for b in range(8):
                base = a * 64 + b * 8
                t_ref[base:base + 8, :] = b0[a:a + 1, :] + b1[b:b + 1, :] + b2
    return pl.pallas_call(
        body,
        out_shape=jax.ShapeDtypeStruct((512, H), F32),
    )(bond_emb)


def _xc_init(xclf, emb, cpad):
    def body(xcl_ref, emb_ref, o_ref):
        oh = (xcl_ref[...] ==
              lax.broadcasted_iota(jnp.int32, (cpad, 4), 1).astype(F32)
              ).astype(F32)
        o_ref[...] = jnp.dot(oh, emb_ref[...], preferred_element_type=F32)
    return pl.pallas_call(
        body,
        out_shape=jax.ShapeDtypeStruct((cpad, H), F32),
    )(xclf, emb)


_BR = 400
_NBLK = N // _BR


def _mlp1(h, aggp, epsb, w1, b1):
    def body(h_ref, agg_ref, eps_ref, w1_ref, b1_ref, y_ref, st_ref):
        i = pl.program_id(0)
        e = eps_ref[0:1, 0:1]
        z = e * h_ref[...] + agg_ref[0] + agg_ref[1]
        y = jnp.dot(z, w1_ref[...], preferred_element_type=F32) + b1_ref[...]
        y_ref[...] = y

        @pl.when(i == 0)
        def _():
            st_ref[...] = jnp.zeros_like(st_ref)
        st_ref[0:1, :] += jnp.sum(y, axis=0, keepdims=True)
        st_ref[1:2, :] += jnp.sum(y * y, axis=0, keepdims=True)

    return pl.pallas_call(
        body,
        grid=(_NBLK,),
        in_specs=[
            pl.BlockSpec((_BR, H), lambda i: (i, 0)),
            pl.BlockSpec((2, _BR, H), lambda i: (0, i, 0)),
            pl.BlockSpec((8, H), lambda i: (0, 0)),
            pl.BlockSpec((H, 2 * H), lambda i: (0, 0)),
            pl.BlockSpec((1, 2 * H), lambda i: (0, 0)),
        ],
        out_specs=[
            pl.BlockSpec((_BR, 2 * H), lambda i: (i, 0)),
            pl.BlockSpec((8, 2 * H), lambda i: (0, 0)),
        ],
        out_shape=[
            jax.ShapeDtypeStruct((N, 2 * H), F32),
            jax.ShapeDtypeStruct((8, 2 * H), F32),
        ],
    )(h, aggp, epsb, w1, b1)


def _mlp2(y1, st1, g1, be1, w2, b2):
    def body(y_ref, st_ref, g_ref, be_ref, w2_ref, b2_ref, o_ref, st2_ref):
        i = pl.program_id(0)
        mean = st_ref[0:1, :] * (1.0 / N)
        var = st_ref[1:2, :] * (1.0 / N) - mean * mean
        inv = lax.rsqrt(var + 1e-5)
        yn = (y_ref[...] - mean) * (inv * g_ref[...]) + be_ref[...]
        yn = jnp.maximum(yn, 0.0)
        o = jnp.dot(yn, w2_ref[...], preferred_element_type=F32) + b2_ref[...]
        o_ref[...] = o

        @pl.when(i == 0)
        def _():
            st2_ref[...] = jnp.zeros_like(st2_ref)
        st2_ref[0:1, :] += jnp.sum(o, axis=0, keepdims=True)
        st2_ref[1:2, :] += jnp.sum(o * o, axis=0, keepdims=True)

    return pl.pallas_call(
        body,
        grid=(_NBLK,),
        in_specs=[
            pl.BlockSpec((_BR, 2 * H), lambda i: (i, 0)),
            pl.BlockSpec((8, 2 * H), lambda i: (0, 0)),
            pl.BlockSpec((1, 2 * H), lambda i: (0, 0)),
            pl.BlockSpec((1, 2 * H), lambda i: (0, 0)),
            pl.BlockSpec((2 * H, H), lambda i: (0, 0)),
            pl.BlockSpec((1, H), lambda i: (0, 0)),
        ],
        out_specs=[
            pl.BlockSpec((_BR, H), lambda i: (i, 0)),
            pl.BlockSpec((8, H), lambda i: (0, 0)),
        ],
        out_shape=[
            jax.ShapeDtypeStruct((N, H), F32),
            jax.ShapeDtypeStruct((8, H), F32),
        ],
    )(y1, st1, g1, be1, w2, b2)


def _mlp3(y2, st2, bg, bb):
    def body(y_ref, st_ref, g_ref, b_ref, o_ref):
        mean = st_ref[0:1, :] * (1.0 / N)
        var = st_ref[1:2, :] * (1.0 / N) - mean * mean
        inv = lax.rsqrt(var + 1e-5)
        hn = (y_ref[...] - mean) * (inv * g_ref[...]) + b_ref[...]
        o_ref[...] = jnp.maximum(hn, 0.0)

    return pl.pallas_call(
        body,
        grid=(_NBLK,),
        in_specs=[
            pl.BlockSpec((_BR, H), lambda i: (i, 0)),
            pl.BlockSpec((8, H), lambda i: (0, 0)),
            pl.BlockSpec((1, H), lambda i: (0, 0)),
            pl.BlockSpec((1, H), lambda i: (0, 0)),
        ],
        out_specs=pl.BlockSpec((_BR, H), lambda i: (i, 0)),
        out_shape=jax.ShapeDtypeStruct((N, H), F32),
    )(y2, st2, bg, bb)


def _a2c_post(xc, csp, cnt, wa, ba, w1, b1, w2, b2, cpad):
    def body(xc_ref, cs_ref, cnt_ref, wa_ref, ba_ref, w1_ref, b1_ref,
             w2_ref, b2_ref, o_ref):
        sm = cs_ref[0] + cs_ref[1]
        cl = jnp.maximum(cnt_ref[:, 0:1], 1.0)
        mean = sm / cl
        t = xc_ref[...] + jnp.maximum(
            jnp.dot(mean, wa_ref[...], preferred_element_type=F32)
            + ba_ref[...], 0.0)
        u = jnp.maximum(
            jnp.dot(t, w1_ref[...], preferred_element_type=F32)
            + b1_ref[...], 0.0)
        o_ref[...] = t + jnp.dot(u, w2_ref[...], preferred_element_type=F32) \
            + b2_ref[...]

    return pl.pallas_call(
        body,
        out_shape=jax.ShapeDtypeStruct((cpad, H), F32),
    )(xc, csp, cnt, wa, ba, w1, b1, w2, b2)


def _c2a_post(h, n5, n6, c5, c6, w5, b5, w6, b6):
    def body(h_ref, n5_ref, n6_ref, c5_ref, c6_ref, w5_ref, b5_ref, w6_ref,
             b6_ref, o_ref):
        def contrib(nref, cref, wref, bref):
            sm = nref[0] + nref[1]
            cl = jnp.maximum(cref[:, 0:1], 1.0)
            mean = sm / cl
            return jnp.maximum(
                jnp.dot(mean, wref[...], preferred_element_type=F32)
                + bref[...], 0.0)
        o_ref[...] = h_ref[...] + contrib(n5_ref, c5_ref, w5_ref, b5_ref) \
            + contrib(n6_ref, c6_ref, w6_ref, b6_ref)

    return pl.pallas_call(
        body,
        grid=(_NBLK,),
        in_specs=[
            pl.BlockSpec((_BR, H), lambda i: (i, 0)),
            pl.BlockSpec((2, _BR, H), lambda i: (0, i, 0)),
            pl.BlockSpec((2, _BR, H), lambda i: (0, i, 0)),
            pl.BlockSpec((_BR, 8), lambda i: (i, 0)),
            pl.BlockSpec((_BR, 8), lambda i: (i, 0)),
            pl.BlockSpec((H, H), lambda i: (0, 0)),
            pl.BlockSpec((1, H), lambda i: (0, 0)),
            pl.BlockSpec((H, H), lambda i: (0, 0)),
            pl.BlockSpec((1, H), lambda i: (0, 0)),
        ],
        out_specs=pl.BlockSpec((_BR, H), lambda i: (i, 0)),
        out_shape=jax.ShapeDtypeStruct((N, H), F32),
    )(h, n5, n6, c5, c6, w5, b5, w6, b6)


def _final_tc(poolp, cntg, aw, ab, lw, lb):
    def body(p_ref, c_ref, aw_ref, ab_ref, lw_ref, lb_ref, o_ref):
        sm = p_ref[0] + p_ref[1]
        cl = jnp.maximum(c_ref[:, 0:1], 1.0)
        g = sm / cl
        g = jnp.maximum(
            jnp.dot(g, aw_ref[...], preferred_element_type=F32)
            + ab_ref[...], 0.0)
        o_ref[...] = jnp.dot(g, lw_ref[...], preferred_element_type=F32) \
            + lb_ref[...]

    return pl.pallas_call(
        body,
        out_shape=jax.ShapeDtypeStruct((NG, H), F32),
    )(poolp, cntg, aw, ab, lw, lb)


def kernel(x, edge_index, edge_attr, batch, x_clique_5_cycle,
           atom2clique_5_cycle, x_clique_6_cycle, atom2clique_6_cycle,
           params):
    p = params
    i32 = jnp.int32
    xcl = {'5': x_clique_5_cycle, '6': x_clique_6_cycle}
    a2c = {'5': atom2clique_5_cycle, '6': atom2clique_6_cycle}

    tabA = p['atom_emb'].reshape(9 * 64, H)
    xind = (x.astype(i32) + (jnp.arange(9, dtype=i32) * 64)[None, :]) \
        .T.reshape(-1)

    src = edge_index[0].astype(i32)
    dst = edge_index[1].astype(i32)
    ea = edge_attr.astype(i32)
    code = ea[:, 0] * 64 + ea[:, 1] * 8 + ea[:, 2]
    epadn = EPAD - E
    srcp = jnp.concatenate([src, jnp.zeros((epadn,), i32)])
    dstp = jnp.concatenate([dst, jnp.full((epadn,), N, i32)])
    codep = jnp.concatenate([code, jnp.zeros((epadn,), i32)])

    pidx = {}
    for kk in ('5', '6'):
        row = a2c[kk][0].astype(i32)
        col = a2c[kk][1].astype(i32)
        pr = PPAD[kk] - row.shape[0]
        pidx[kk] = dict(
            a2c_g=jnp.concatenate([row, jnp.zeros((pr,), i32)]),
            a2c_s=jnp.concatenate([col, jnp.full((pr,), CSZ[kk], i32)]),
            c2a_g=jnp.concatenate([col, jnp.zeros((pr,), i32)]),
            c2a_s=jnp.concatenate([row, jnp.full((pr,), N, i32)]),
        )

    xcf = {}
    for kk in ('5', '6'):
        xp = jnp.concatenate(
            [xcl[kk].astype(i32),
             jnp.zeros((CPAD[kk] - CSZ[kk],), i32)])
        xcf[kk] = xp.astype(F32)[:, None]

    batch_i = batch.astype(i32)

    cnt_a2c = {kk: _cnt_tc(_hist_sc(pidx[kk]['a2c_s'], CPAD[kk], KC,
                                    PPAD[kk] // KC), CPAD[kk])
               for kk in ('5', '6')}
    cnt_c2a = {kk: _cnt_tc(_hist_sc(pidx[kk]['c2a_s'], NP, KC,
                                    PPAD[kk] // KC), NP)
               for kk in ('5', '6')}
    cnt_g = _cnt_tc(_hist_sc(batch_i, NG, KS, N // KS), NG)

    h = _embed_sc(tabA, xind)
    xc = {kk: _xc_init(xcf[kk], p['cycle_emb'][kk], CPAD[kk])
          for kk in ('5', '6')}

    for li in range(len(p['layers'])):
        lp = p['layers'][li]
        nn = lp['nn']
        tbl = _tbuild(lp['bond_emb'])
        aggp = _edge_sc(h, tbl, srcp, dstp, codep)
        epsb = jnp.full((8, H), 1.0, F32) * (1.0 + lp['eps'])
        y1, st1 = _mlp1(h, aggp, epsb, nn['W1'], nn['b1'].reshape(1, -1))
        y2, st2 = _mlp2(y1, st1, nn['g1'].reshape(1, -1),
                        nn['be1'].reshape(1, -1), nn['W2'],
                        nn['b2'].reshape(1, -1))
        h = _mlp3(y2, st2, lp['bn']['g'].reshape(1, -1),
                  lp['bn']['b'].reshape(1, -1))
        for kk in ('5', '6'):
            csp = _gs_sc(h, pidx[kk]['a2c_g'], pidx[kk]['a2c_s'],
                         CPAD[kk], PCH[kk])
            cw = lp['cyc'][kk]
            xc[kk] = _a2c_post(
                xc[kk], csp, cnt_a2c[kk],
                lp['a2c'][kk]['W'], lp['a2c'][kk]['b'].reshape(1, -1),
                cw['l1']['W'], cw['l1']['b'].reshape(1, -1),
                cw['l2']['W'], cw['l2']['b'].reshape(1, -1),
                CPAD[kk])
        nsp = {kk: _gs_sc(xc[kk], pidx[kk]['c2a_g'], pidx[kk]['c2a_s'],
                          NP, PCH[kk])
               for kk in ('5', '6')}
        h = _c2a_post(h, nsp['5'], nsp['6'], cnt_c2a['5'], cnt_c2a['6'],
                      lp['c2a']['5']['W'], lp['c2a']['5']['b'].reshape(1, -1),
                      lp['c2a']['6']['W'], lp['c2a']['6']['b'].reshape(1, -1))

    poolp = _pool_sc(h, batch_i)
    return _final_tc(poolp, cnt_g, p['atom_lin']['W'],
                     p['atom_lin']['b'].reshape(1, -1),
                     p['lin']['W'], p['lin']['b'].reshape(1, -1))


# speedup vs baseline: 3.2223x; 1.0655x over previous
import functools

import jax
import jax.numpy as jnp
from jax import lax
from jax.experimental import pallas as pl
from jax.experimental.pallas import tpu as pltpu
from jax.experimental.pallas import tpu_sc as plsc

H = 128
N = 10000
NP = 10112
NG = 512
E = 320000
KC = 128
KS = 80
NWORK = 32
ECH = 80
EPAD = NWORK * ECH * KC
CSZ = {'5': 2000, '6': 2500}
CPAD = {'5': 2048, '6': 2560}
PCH = {'5': 3, '6': 4}
PPAD = {'5': NWORK * 3 * KC, '6': NWORK * 4 * KC}
ZR = 128
F32 = jnp.float32

_MESH = plsc.VectorSubcoreMesh(core_axis_name="c", subcore_axis_name="s")


def _fill_const(zbuf, rows, vec16s):
    def body(i, _):
        for j in range(H // 16):
            zbuf[i, pl.ds(16 * j, 16)] = vec16s[j]
        return 0
    lax.fori_loop(0, rows, body, 0)


def _zero_stripe(acc, zbuf, stripe, sid):
    base = sid * stripe
    nfull = stripe // ZR
    rem = stripe % ZR
    for q in range(nfull):
        pltpu.sync_copy(zbuf, acc.at[pl.ds(base + q * ZR, ZR)])
    if rem:
        pltpu.sync_copy(zbuf.at[pl.ds(0, rem)],
                        acc.at[pl.ds(base + nfull * ZR, rem)])


def _zeros16():
    return [jnp.zeros((16,), F32)] * (H // 16)


def _edge_sc(h, tbl, pack):
    @functools.partial(
        pl.kernel, mesh=_MESH,
        out_type=jax.ShapeDtypeStruct((2, NP, H), F32),
        scratch_types=[
            pltpu.VMEM((3, KC), jnp.int32),
            pltpu.VMEM((KC, H), F32),
            pltpu.VMEM((KC, H), F32),
            pltpu.VMEM((ZR, H), F32),
            pltpu.VMEM_SHARED((NP, H), F32),
            pltpu.SemaphoreType.DMA,
            pltpu.SemaphoreType.DMA,
        ],
    )
    def k(h_hbm, t_hbm, pack_hbm, out_hbm, packa, hra, tra,
          zbuf, acc, semh, semt):
        c = lax.axis_index("c")
        s = lax.axis_index("s")
        w = c * 16 + s
        _fill_const(zbuf, ZR, _zeros16())
        stripe = NP // 16
        _zero_stripe(acc, zbuf, stripe, s)
        plsc.subcore_barrier()
        cbase = w * ECH

        def valu_scatter(pk, hr, tr):
            def vrow(i, _):
                for j in range(H // 16):
                    sl = pl.ds(16 * j, 16)
                    tr[i, sl] = jnp.maximum(hr[i, sl] + tr[i, sl], 0.0)
                return 0
            lax.fori_loop(0, KC, vrow, 0)
            pltpu.sync_copy(tr, acc.at[pk.at[1]], add=True)

        def body(g, _):
            pltpu.sync_copy(pack_hbm.at[cbase + g], packa)
            ca_h = pltpu.async_copy(h_hbm.at[packa.at[0]], hra, semh)
            ca_t = pltpu.async_copy(t_hbm.at[packa.at[2]], tra, semt)
            ca_h.wait()
            ca_t.wait()
            valu_scatter(packa, hra, tra)
            return 0
        lax.fori_loop(0, ECH, body, 0)
        plsc.subcore_barrier()
        pltpu.sync_copy(acc.at[pl.ds(s * stripe, stripe)],
                        out_hbm.at[c, pl.ds(s * stripe, stripe)])

    return k(h, tbl, pack)


def _gs_sc(tab, gidx, sidx, acc_rows, nchunks):
    @functools.partial(
        pl.kernel, mesh=_MESH,
        out_type=jax.ShapeDtypeStruct((2, acc_rows, H), F32),
        scratch_types=[
            pltpu.VMEM((KC,), jnp.int32),
            pltpu.VMEM((KC,), jnp.int32),
            pltpu.VMEM((KC, H), F32),
            pltpu.VMEM((ZR, H), F32),
            pltpu.VMEM_SHARED((acc_rows, H), F32),
            pltpu.SemaphoreType.DMA,
        ],
    )
    def k(tab_hbm, gidx_hbm, sidx_hbm, out_hbm, gbuf, sbuf, rows, zbuf, acc,
          sem):
        c = lax.axis_index("c")
        s = lax.axis_index("s")
        w = c * 16 + s
        _fill_const(zbuf, ZR, _zeros16())
        stripe = acc_rows // 16
        _zero_stripe(acc, zbuf, stripe, s)
        plsc.subcore_barrier()

        def chunk(g, _):
            base = w * (nchunks * KC) + g * KC
            pltpu.sync_copy(gidx_hbm.at[pl.ds(base, KC)], gbuf)
            pltpu.sync_copy(sidx_hbm.at[pl.ds(base, KC)], sbuf)
            pltpu.async_copy(tab_hbm.at[gbuf], rows, sem).wait()
            pltpu.sync_copy(rows, acc.at[sbuf], add=True)
            return 0
        lax.fori_loop(0, nchunks, chunk, 0)
        plsc.subcore_barrier()
        pltpu.sync_copy(acc.at[pl.ds(s * stripe, stripe)],
                        out_hbm.at[c, pl.ds(s * stripe, stripe)])

    return k(tab, gidx, sidx)


def _hist_sc(sidx, acc_rows, kchunk, nchunks_total):
    nloop = -(-nchunks_total // NWORK)

    @functools.partial(
        pl.kernel, mesh=_MESH,
        out_type=jax.ShapeDtypeStruct((2, acc_rows, H), F32),
        scratch_types=[
            pltpu.VMEM((kchunk,), jnp.int32),
            pltpu.VMEM((kchunk, H), F32),
            pltpu.VMEM((ZR, H), F32),
            pltpu.VMEM_SHARED((acc_rows, H), F32),
        ],
    )
    def k(sidx_hbm, out_hbm, sbuf, rows, zbuf, acc):
        c = lax.axis_index("c")
        s = lax.axis_index("s")
        w = c * 16 + s
        _fill_const(zbuf, ZR, _zeros16())
        stripe = acc_rows // 16
        _zero_stripe(acc, zbuf, stripe, s)
        lane = lax.iota(jnp.int32, 16)
        one16 = jnp.where(lane == 0, jnp.float32(1.0), jnp.float32(0.0))
        _fill_const(rows, kchunk, [one16] + _zeros16()[1:])
        plsc.subcore_barrier()
        for t in range(nloop):
            j = w + NWORK * t

            @pl.when(j < nchunks_total)
            def _():
                base = j * kchunk
                pltpu.sync_copy(sidx_hbm.at[pl.ds(base, kchunk)], sbuf)
                pltpu.sync_copy(rows, acc.at[sbuf], add=True)
        plsc.subcore_barrier()
        pltpu.sync_copy(acc.at[pl.ds(s * stripe, stripe)],
                        out_hbm.at[c, pl.ds(s * stripe, stripe)])

    return k(sidx)


def _embed_sc(tabA, xind):
    @functools.partial(
        pl.kernel, mesh=_MESH,
        out_type=jax.ShapeDtypeStruct((N, H), F32),
        scratch_types=[
            pltpu.VMEM((KS,), jnp.int32),
            pltpu.VMEM((KS, H), F32),
            pltpu.VMEM((KS, H), F32),
            pltpu.SemaphoreType.DMA,
        ],
    )
    def k(tab_hbm, xind_hbm, out_hbm, idxb, rows, accb, sem):
        c = lax.axis_index("c")
        s = lax.axis_index("s")
        w = c * 16 + s
        for t in range(4):
            j = w + NWORK * t

            @pl.when(j < N // KS)
            def _():
                base = j * KS
                _fill_const(accb, KS, _zeros16())
                for f in range(9):
                    pltpu.sync_copy(xind_hbm.at[pl.ds(f * N + base, KS)],
                                    idxb)
                    pltpu.async_copy(tab_hbm.at[idxb], rows, sem).wait()

                    def addrow(i, _):
                        for jj in range(H // 16):
                            sl = pl.ds(16 * jj, 16)
                            accb[i, sl] = accb[i, sl] + rows[i, sl]
                        return 0
                    lax.fori_loop(0, KS, addrow, 0)
                pltpu.sync_copy(accb, out_hbm.at[pl.ds(base, KS)])

    return k(tabA, xind)


def _pool_sc(h, batch):
    @functools.partial(
        pl.kernel, mesh=_MESH,
        out_type=jax.ShapeDtypeStruct((2, NG, H), F32),
        scratch_types=[
            pltpu.VMEM((KS,), jnp.int32),
            pltpu.VMEM((KS, H), F32),
            pltpu.VMEM((ZR, H), F32),
            pltpu.VMEM_SHARED((NG, H), F32),
        ],
    )
    def k(h_hbm, b_hbm, out_hbm, idxb, rows, zbuf, acc):
        c = lax.axis_index("c")
        s = lax.axis_index("s")
        w = c * 16 + s
        _fill_const(zbuf, ZR, _zeros16())
        stripe = NG // 16
        _zero_stripe(acc, zbuf, stripe, s)
        plsc.subcore_barrier()
        for t in range(4):
            j = w + NWORK * t

            @pl.when(j < N // KS)
            def _():
                base = j * KS
                pltpu.sync_copy(b_hbm.at[pl.ds(base, KS)], idxb)
                pltpu.sync_copy(h_hbm.at[pl.ds(base, KS)], rows)
                pltpu.sync_copy(rows, acc.at[idxb], add=True)
        plsc.subcore_barrier()
        pltpu.sync_copy(acc.at[pl.ds(s * stripe, stripe)],
                        out_hbm.at[c, pl.ds(s * stripe, stripe)])

    return k(h, batch)


def _cnt_tc(hist, rows):
    def body(h_ref, o_ref):
        sm = h_ref[0][:, 0:1] + h_ref[1][:, 0:1]
        o_ref[...] = jnp.broadcast_to(sm, (rows, 8))
    return pl.pallas_call(
        body,
        out_shape=jax.ShapeDtypeStruct((rows, 8), F32),
    )(hist)


def _tbuild(bond_emb):
    def body(be_ref, t_ref):
        b0 = be_ref[0]
        b1 = be_ref[1]
        b2 = be_ref[2]
        for a in range(8):
            for b in range(8):
                base = a * 64 + b * 8
                t_ref[base:base + 8, :] = b0[a:a + 1, :] + b1[b:b + 1, :] + b2
    return pl.pallas_call(
        body,
        out_shape=jax.ShapeDtypeStruct((512, H), F32),
    )(bond_emb)


def _xc_init(xclf, emb, cpad):
    def body(xcl_ref, emb_ref, o_ref):
        oh = (xcl_ref[...] ==
              lax.broadcasted_iota(jnp.int32, (cpad, 4), 1).astype(F32)
              ).astype(F32)
        o_ref[...] = jnp.dot(oh, emb_ref[...], preferred_element_type=F32)
    return pl.pallas_call(
        body,
        out_shape=jax.ShapeDtypeStruct((cpad, H), F32),
    )(xclf, emb)


_BR = 400
_NBLK = N // _BR


def _mlp1(h, aggp, epsb, w1, b1):
    def body(h_ref, agg_ref, eps_ref, w1_ref, b1_ref, y_ref, st_ref):
        i = pl.program_id(0)
        e = eps_ref[0:1, 0:1]
        z = e * h_ref[...] + agg_ref[0] + agg_ref[1]
        y = jnp.dot(z, w1_ref[...], preferred_element_type=F32) + b1_ref[...]
        y_ref[...] = y

        @pl.when(i == 0)
        def _():
            st_ref[...] = jnp.zeros_like(st_ref)
        st_ref[0:1, :] += jnp.sum(y, axis=0, keepdims=True)
        st_ref[1:2, :] += jnp.sum(y * y, axis=0, keepdims=True)

    return pl.pallas_call(
        body,
        grid=(_NBLK,),
        in_specs=[
            pl.BlockSpec((_BR, H), lambda i: (i, 0)),
            pl.BlockSpec((2, _BR, H), lambda i: (0, i, 0)),
            pl.BlockSpec((8, H), lambda i: (0, 0)),
            pl.BlockSpec((H, 2 * H), lambda i: (0, 0)),
            pl.BlockSpec((1, 2 * H), lambda i: (0, 0)),
        ],
        out_specs=[
            pl.BlockSpec((_BR, 2 * H), lambda i: (i, 0)),
            pl.BlockSpec((8, 2 * H), lambda i: (0, 0)),
        ],
        out_shape=[
            jax.ShapeDtypeStruct((N, 2 * H), F32),
            jax.ShapeDtypeStruct((8, 2 * H), F32),
        ],
    )(h, aggp, epsb, w1, b1)


def _mlp2(y1, st1, g1, be1, w2, b2):
    def body(y_ref, st_ref, g_ref, be_ref, w2_ref, b2_ref, o_ref, st2_ref):
        i = pl.program_id(0)
        mean = st_ref[0:1, :] * (1.0 / N)
        var = st_ref[1:2, :] * (1.0 / N) - mean * mean
        inv = lax.rsqrt(var + 1e-5)
        yn = (y_ref[...] - mean) * (inv * g_ref[...]) + be_ref[...]
        yn = jnp.maximum(yn, 0.0)
        o = jnp.dot(yn, w2_ref[...], preferred_element_type=F32) + b2_ref[...]
        o_ref[...] = o

        @pl.when(i == 0)
        def _():
            st2_ref[...] = jnp.zeros_like(st2_ref)
        st2_ref[0:1, :] += jnp.sum(o, axis=0, keepdims=True)
        st2_ref[1:2, :] += jnp.sum(o * o, axis=0, keepdims=True)

    return pl.pallas_call(
        body,
        grid=(_NBLK,),
        in_specs=[
            pl.BlockSpec((_BR, 2 * H), lambda i: (i, 0)),
            pl.BlockSpec((8, 2 * H), lambda i: (0, 0)),
            pl.BlockSpec((1, 2 * H), lambda i: (0, 0)),
            pl.BlockSpec((1, 2 * H), lambda i: (0, 0)),
            pl.BlockSpec((2 * H, H), lambda i: (0, 0)),
            pl.BlockSpec((1, H), lambda i: (0, 0)),
        ],
        out_specs=[
            pl.BlockSpec((_BR, H), lambda i: (i, 0)),
            pl.BlockSpec((8, H), lambda i: (0, 0)),
        ],
        out_shape=[
            jax.ShapeDtypeStruct((N, H), F32),
            jax.ShapeDtypeStruct((8, H), F32),
        ],
    )(y1, st1, g1, be1, w2, b2)


def _mlp3(y2, st2, bg, bb):
    def body(y_ref, st_ref, g_ref, b_ref, o_ref):
        mean = st_ref[0:1, :] * (1.0 / N)
        var = st_ref[1:2, :] * (1.0 / N) - mean * mean
        inv = lax.rsqrt(var + 1e-5)
        hn = (y_ref[...] - mean) * (inv * g_ref[...]) + b_ref[...]
        o_ref[...] = jnp.maximum(hn, 0.0)

    return pl.pallas_call(
        body,
        grid=(_NBLK,),
        in_specs=[
            pl.BlockSpec((_BR, H), lambda i: (i, 0)),
            pl.BlockSpec((8, H), lambda i: (0, 0)),
            pl.BlockSpec((1, H), lambda i: (0, 0)),
            pl.BlockSpec((1, H), lambda i: (0, 0)),
        ],
        out_specs=pl.BlockSpec((_BR, H), lambda i: (i, 0)),
        out_shape=jax.ShapeDtypeStruct((N, H), F32),
    )(y2, st2, bg, bb)


def _a2c_post(xc, csp, cnt, wa, ba, w1, b1, w2, b2, cpad):
    def body(xc_ref, cs_ref, cnt_ref, wa_ref, ba_ref, w1_ref, b1_ref,
             w2_ref, b2_ref, o_ref):
        sm = cs_ref[0] + cs_ref[1]
        cl = jnp.maximum(cnt_ref[:, 0:1], 1.0)
        mean = sm / cl
        t = xc_ref[...] + jnp.maximum(
            jnp.dot(mean, wa_ref[...], preferred_element_type=F32)
            + ba_ref[...], 0.0)
        u = jnp.maximum(
            jnp.dot(t, w1_ref[...], preferred_element_type=F32)
            + b1_ref[...], 0.0)
        o_ref[...] = t + jnp.dot(u, w2_ref[...], preferred_element_type=F32) \
            + b2_ref[...]

    return pl.pallas_call(
        body,
        out_shape=jax.ShapeDtypeStruct((cpad, H), F32),
    )(xc, csp, cnt, wa, ba, w1, b1, w2, b2)


def _c2a_post(h, n5, n6, c5, c6, w5, b5, w6, b6):
    def body(h_ref, n5_ref, n6_ref, c5_ref, c6_ref, w5_ref, b5_ref, w6_ref,
             b6_ref, o_ref):
        def contrib(nref, cref, wref, bref):
            sm = nref[0] + nref[1]
            cl = jnp.maximum(cref[:, 0:1], 1.0)
            mean = sm / cl
            return jnp.maximum(
                jnp.dot(mean, wref[...], preferred_element_type=F32)
                + bref[...], 0.0)
        o_ref[...] = h_ref[...] + contrib(n5_ref, c5_ref, w5_ref, b5_ref) \
            + contrib(n6_ref, c6_ref, w6_ref, b6_ref)

    return pl.pallas_call(
        body,
        grid=(_NBLK,),
        in_specs=[
            pl.BlockSpec((_BR, H), lambda i: (i, 0)),
            pl.BlockSpec((2, _BR, H), lambda i: (0, i, 0)),
            pl.BlockSpec((2, _BR, H), lambda i: (0, i, 0)),
            pl.BlockSpec((_BR, 8), lambda i: (i, 0)),
            pl.BlockSpec((_BR, 8), lambda i: (i, 0)),
            pl.BlockSpec((H, H), lambda i: (0, 0)),
            pl.BlockSpec((1, H), lambda i: (0, 0)),
            pl.BlockSpec((H, H), lambda i: (0, 0)),
            pl.BlockSpec((1, H), lambda i: (0, 0)),
        ],
        out_specs=pl.BlockSpec((_BR, H), lambda i: (i, 0)),
        out_shape=jax.ShapeDtypeStruct((N, H), F32),
    )(h, n5, n6, c5, c6, w5, b5, w6, b6)


def _final_tc(poolp, cntg, aw, ab, lw, lb):
    def body(p_ref, c_ref, aw_ref, ab_ref, lw_ref, lb_ref, o_ref):
        sm = p_ref[0] + p_ref[1]
        cl = jnp.maximum(c_ref[:, 0:1], 1.0)
        g = sm / cl
        g = jnp.maximum(
            jnp.dot(g, aw_ref[...], preferred_element_type=F32)
            + ab_ref[...], 0.0)
        o_ref[...] = jnp.dot(g, lw_ref[...], preferred_element_type=F32) \
            + lb_ref[...]

    return pl.pallas_call(
        body,
        out_shape=jax.ShapeDtypeStruct((NG, H), F32),
    )(poolp, cntg, aw, ab, lw, lb)


def kernel(x, edge_index, edge_attr, batch, x_clique_5_cycle,
           atom2clique_5_cycle, x_clique_6_cycle, atom2clique_6_cycle,
           params):
    p = params
    i32 = jnp.int32
    xcl = {'5': x_clique_5_cycle, '6': x_clique_6_cycle}
    a2c = {'5': atom2clique_5_cycle, '6': atom2clique_6_cycle}

    tabA = p['atom_emb'].reshape(9 * 64, H)
    xind = (x.astype(i32) + (jnp.arange(9, dtype=i32) * 64)[None, :]) \
        .T.reshape(-1)

    src = edge_index[0].astype(i32)
    dst = edge_index[1].astype(i32)
    ea = edge_attr.astype(i32)
    code = ea[:, 0] * 64 + ea[:, 1] * 8 + ea[:, 2]
    epadn = EPAD - E
    srcp = jnp.concatenate([src, jnp.zeros((epadn,), i32)])
    dstp = jnp.concatenate([dst, jnp.full((epadn,), N, i32)])
    codep = jnp.concatenate([code, jnp.zeros((epadn,), i32)])
    epack = jnp.stack([srcp.reshape(-1, KC), dstp.reshape(-1, KC),
                       codep.reshape(-1, KC)], axis=1)

    pidx = {}
    for kk in ('5', '6'):
        row = a2c[kk][0].astype(i32)
        col = a2c[kk][1].astype(i32)
        pr = PPAD[kk] - row.shape[0]
        pidx[kk] = dict(
            a2c_g=jnp.concatenate([row, jnp.zeros((pr,), i32)]),
            a2c_s=jnp.concatenate([col, jnp.full((pr,), CSZ[kk], i32)]),
            c2a_g=jnp.concatenate([col, jnp.zeros((pr,), i32)]),
            c2a_s=jnp.concatenate([row, jnp.full((pr,), N, i32)]),
        )

    xcf = {}
    for kk in ('5', '6'):
        xp = jnp.concatenate(
            [xcl[kk].astype(i32),
             jnp.zeros((CPAD[kk] - CSZ[kk],), i32)])
        xcf[kk] = xp.astype(F32)[:, None]

    batch_i = batch.astype(i32)

    cnt_a2c = {kk: _cnt_tc(_hist_sc(pidx[kk]['a2c_s'], CPAD[kk], KC,
                                    PPAD[kk] // KC), CPAD[kk])
               for kk in ('5', '6')}
    cnt_c2a = {kk: _cnt_tc(_hist_sc(pidx[kk]['c2a_s'], NP, KC,
                                    PPAD[kk] // KC), NP)
               for kk in ('5', '6')}
    cnt_g = _cnt_tc(_hist_sc(batch_i, NG, KS, N // KS), NG)

    h = _embed_sc(tabA, xind)
    xc = {kk: _xc_init(xcf[kk], p['cycle_emb'][kk], CPAD[kk])
          for kk in ('5', '6')}

    for li in range(len(p['layers'])):
        lp = p['layers'][li]
        nn = lp['nn']
        tbl = _tbuild(lp['bond_emb'])
        aggp = _edge_sc(h, tbl, epack)
        epsb = jnp.full((8, H), 1.0, F32) * (1.0 + lp['eps'])
        y1, st1 = _mlp1(h, aggp, epsb, nn['W1'], nn['b1'].reshape(1, -1))
        y2, st2 = _mlp2(y1, st1, nn['g1'].reshape(1, -1),
                        nn['be1'].reshape(1, -1), nn['W2'],
                        nn['b2'].reshape(1, -1))
        h = _mlp3(y2, st2, lp['bn']['g'].reshape(1, -1),
                  lp['bn']['b'].reshape(1, -1))
        for kk in ('5', '6'):
            csp = _gs_sc(h, pidx[kk]['a2c_g'], pidx[kk]['a2c_s'],
                         CPAD[kk], PCH[kk])
            cw = lp['cyc'][kk]
            xc[kk] = _a2c_post(
                xc[kk], csp, cnt_a2c[kk],
                lp['a2c'][kk]['W'], lp['a2c'][kk]['b'].reshape(1, -1),
                cw['l1']['W'], cw['l1']['b'].reshape(1, -1),
                cw['l2']['W'], cw['l2']['b'].reshape(1, -1),
                CPAD[kk])
        nsp = {kk: _gs_sc(xc[kk], pidx[kk]['c2a_g'], pidx[kk]['c2a_s'],
                          NP, PCH[kk])
               for kk in ('5', '6')}
        h = _c2a_post(h, nsp['5'], nsp['6'], cnt_c2a['5'], cnt_c2a['6'],
                      lp['c2a']['5']['W'], lp['c2a']['5']['b'].reshape(1, -1),
                      lp['c2a']['6']['W'], lp['c2a']['6']['b'].reshape(1, -1))

    poolp = _pool_sc(h, batch_i)
    return _final_tc(poolp, cnt_g, p['atom_lin']['W'],
                     p['atom_lin']['b'].reshape(1, -1),
                     p['lin']['W'], p['lin']['b'].reshape(1, -1))


# speedup vs baseline: 3.2754x; 1.0165x over previous
import functools

import jax
import jax.numpy as jnp
from jax import lax
from jax.experimental import pallas as pl
from jax.experimental.pallas import tpu as pltpu
from jax.experimental.pallas import tpu_sc as plsc

H = 128
N = 10000
NP = 10112
NG = 512
E = 320000
KC = 128
KS = 80
NWORK = 32
ECH = 80
EPAD = NWORK * ECH * KC
CSZ = {'5': 2000, '6': 2500}
CPAD = {'5': 2048, '6': 2560}
PCH = {'5': 3, '6': 4}
PPAD = {'5': NWORK * 3 * KC, '6': NWORK * 4 * KC}
ZR = 128
F32 = jnp.float32

_MESH = plsc.VectorSubcoreMesh(core_axis_name="c", subcore_axis_name="s")


def _fill_const(zbuf, rows, vec16s):
    def body(i, _):
        for j in range(H // 16):
            zbuf[i, pl.ds(16 * j, 16)] = vec16s[j]
        return 0
    lax.fori_loop(0, rows, body, 0)


def _zero_stripe(acc, zbuf, stripe, sid):
    base = sid * stripe
    nfull = stripe // ZR
    rem = stripe % ZR
    for q in range(nfull):
        pltpu.sync_copy(zbuf, acc.at[pl.ds(base + q * ZR, ZR)])
    if rem:
        pltpu.sync_copy(zbuf.at[pl.ds(0, rem)],
                        acc.at[pl.ds(base + nfull * ZR, rem)])


def _zeros16():
    return [jnp.zeros((16,), F32)] * (H // 16)


def _edge_sc(h, tbl, pack):
    @functools.partial(
        pl.kernel, mesh=_MESH,
        out_type=jax.ShapeDtypeStruct((2, NP, H), F32),
        scratch_types=[
            pltpu.VMEM((3, KC), jnp.int32),
            pltpu.VMEM((KC, H), F32),
            pltpu.VMEM((KC, H), F32),
            pltpu.VMEM((ZR, H), F32),
            pltpu.VMEM_SHARED((NP, H), F32),
            pltpu.SemaphoreType.DMA,
            pltpu.SemaphoreType.DMA,
        ],
    )
    def k(h_hbm, t_hbm, pack_hbm, out_hbm, packa, hra, tra,
          zbuf, acc, semh, semt):
        c = lax.axis_index("c")
        s = lax.axis_index("s")
        w = c * 16 + s
        _fill_const(zbuf, ZR, _zeros16())
        stripe = NP // 16
        _zero_stripe(acc, zbuf, stripe, s)
        plsc.subcore_barrier()
        cbase = w * ECH

        def valu_scatter(pk, hr, tr):
            def vrow(i, _):
                for j in range(H // 16):
                    sl = pl.ds(16 * j, 16)
                    tr[i, sl] = jnp.maximum(hr[i, sl] + tr[i, sl], 0.0)
                return 0
            lax.fori_loop(0, KC, vrow, 0)
            pltpu.sync_copy(tr, acc.at[pk.at[1]], add=True)

        def body(g, _):
            pltpu.sync_copy(pack_hbm.at[cbase + g], packa)
            ca_h = pltpu.async_copy(h_hbm.at[packa.at[0]], hra, semh)
            ca_t = pltpu.async_copy(t_hbm.at[packa.at[2]], tra, semt)
            ca_h.wait()
            ca_t.wait()
            valu_scatter(packa, hra, tra)
            return 0
        lax.fori_loop(0, ECH, body, 0)
        plsc.subcore_barrier()
        pltpu.sync_copy(acc.at[pl.ds(s * stripe, stripe)],
                        out_hbm.at[c, pl.ds(s * stripe, stripe)])

    return k(h, tbl, pack)


def _gs_sc(tab, gidx, sidx, acc_rows, nchunks):
    @functools.partial(
        pl.kernel, mesh=_MESH,
        out_type=jax.ShapeDtypeStruct((2, acc_rows, H), F32),
        scratch_types=[
            pltpu.VMEM((KC,), jnp.int32),
            pltpu.VMEM((KC,), jnp.int32),
            pltpu.VMEM((KC, H), F32),
            pltpu.VMEM((ZR, H), F32),
            pltpu.VMEM_SHARED((acc_rows, H), F32),
            pltpu.SemaphoreType.DMA,
        ],
    )
    def k(tab_hbm, gidx_hbm, sidx_hbm, out_hbm, gbuf, sbuf, rows, zbuf, acc,
          sem):
        c = lax.axis_index("c")
        s = lax.axis_index("s")
        w = c * 16 + s
        _fill_const(zbuf, ZR, _zeros16())
        stripe = acc_rows // 16
        _zero_stripe(acc, zbuf, stripe, s)
        plsc.subcore_barrier()

        def chunk(g, _):
            base = w * (nchunks * KC) + g * KC
            pltpu.sync_copy(gidx_hbm.at[pl.ds(base, KC)], gbuf)
            pltpu.sync_copy(sidx_hbm.at[pl.ds(base, KC)], sbuf)
            pltpu.async_copy(tab_hbm.at[gbuf], rows, sem).wait()
            pltpu.sync_copy(rows, acc.at[sbuf], add=True)
            return 0
        lax.fori_loop(0, nchunks, chunk, 0)
        plsc.subcore_barrier()
        pltpu.sync_copy(acc.at[pl.ds(s * stripe, stripe)],
                        out_hbm.at[c, pl.ds(s * stripe, stripe)])

    return k(tab, gidx, sidx)


def _a2c_pair_sc(h, pack5, pack6):
    @functools.partial(
        pl.kernel, mesh=_MESH,
        out_type=[jax.ShapeDtypeStruct((2, CPAD['5'], H), F32),
                  jax.ShapeDtypeStruct((2, CPAD['6'], H), F32)],
        scratch_types=[
            pltpu.VMEM((2, KC), jnp.int32),
            pltpu.VMEM((KC, H), F32),
            pltpu.VMEM((ZR, H), F32),
            pltpu.VMEM_SHARED((CPAD['5'], H), F32),
            pltpu.VMEM_SHARED((CPAD['6'], H), F32),
            pltpu.SemaphoreType.DMA,
        ],
    )
    def k(h_hbm, p5_hbm, p6_hbm, out5_hbm, out6_hbm, pk, rows, zbuf,
          acc5, acc6, sem):
        c = lax.axis_index("c")
        s = lax.axis_index("s")
        w = c * 16 + s
        _fill_const(zbuf, ZR, _zeros16())

        def phase(pack_hbm, out_hbm, acc, acc_rows, nchunks):
            stripe = acc_rows // 16
            _zero_stripe(acc, zbuf, stripe, s)
            plsc.subcore_barrier()
            cbase = w * nchunks
            for g in range(nchunks):
                pltpu.sync_copy(pack_hbm.at[cbase + g], pk)
                pltpu.async_copy(h_hbm.at[pk.at[0]], rows, sem).wait()
                pltpu.sync_copy(rows, acc.at[pk.at[1]], add=True)
            plsc.subcore_barrier()
            pltpu.sync_copy(acc.at[pl.ds(s * stripe, stripe)],
                            out_hbm.at[c, pl.ds(s * stripe, stripe)])
            plsc.subcore_barrier()

        phase(p5_hbm, out5_hbm, acc5, CPAD['5'], PCH['5'])
        phase(p6_hbm, out6_hbm, acc6, CPAD['6'], PCH['6'])

    return k(h, pack5, pack6)


def _c2a_pair_sc(xc5, pack5, xc6, pack6):
    @functools.partial(
        pl.kernel, mesh=_MESH,
        out_type=[jax.ShapeDtypeStruct((2, NP, H), F32),
                  jax.ShapeDtypeStruct((2, NP, H), F32)],
        scratch_types=[
            pltpu.VMEM((2, KC), jnp.int32),
            pltpu.VMEM((KC, H), F32),
            pltpu.VMEM((ZR, H), F32),
            pltpu.VMEM_SHARED((NP, H), F32),
            pltpu.SemaphoreType.DMA,
        ],
    )
    def k(x5_hbm, p5_hbm, x6_hbm, p6_hbm, out5_hbm, out6_hbm, pk, rows,
          zbuf, acc, sem):
        c = lax.axis_index("c")
        s = lax.axis_index("s")
        w = c * 16 + s
        _fill_const(zbuf, ZR, _zeros16())
        stripe = NP // 16

        def phase(tab_hbm, pack_hbm, out_hbm, nchunks):
            _zero_stripe(acc, zbuf, stripe, s)
            plsc.subcore_barrier()
            cbase = w * nchunks
            for g in range(nchunks):
                pltpu.sync_copy(pack_hbm.at[cbase + g], pk)
                pltpu.async_copy(tab_hbm.at[pk.at[0]], rows, sem).wait()
                pltpu.sync_copy(rows, acc.at[pk.at[1]], add=True)
            plsc.subcore_barrier()
            pltpu.sync_copy(acc.at[pl.ds(s * stripe, stripe)],
                            out_hbm.at[c, pl.ds(s * stripe, stripe)])
            plsc.subcore_barrier()

        phase(x5_hbm, p5_hbm, out5_hbm, PCH['5'])
        phase(x6_hbm, p6_hbm, out6_hbm, PCH['6'])

    return k(xc5, pack5, xc6, pack6)


def _hist_sc(sidx, acc_rows, kchunk, nchunks_total):
    nloop = -(-nchunks_total // NWORK)

    @functools.partial(
        pl.kernel, mesh=_MESH,
        out_type=jax.ShapeDtypeStruct((2, acc_rows, H), F32),
        scratch_types=[
            pltpu.VMEM((kchunk,), jnp.int32),
            pltpu.VMEM((kchunk, H), F32),
            pltpu.VMEM((ZR, H), F32),
            pltpu.VMEM_SHARED((acc_rows, H), F32),
        ],
    )
    def k(sidx_hbm, out_hbm, sbuf, rows, zbuf, acc):
        c = lax.axis_index("c")
        s = lax.axis_index("s")
        w = c * 16 + s
        _fill_const(zbuf, ZR, _zeros16())
        stripe = acc_rows // 16
        _zero_stripe(acc, zbuf, stripe, s)
        lane = lax.iota(jnp.int32, 16)
        one16 = jnp.where(lane == 0, jnp.float32(1.0), jnp.float32(0.0))
        _fill_const(rows, kchunk, [one16] + _zeros16()[1:])
        plsc.subcore_barrier()
        for t in range(nloop):
            j = w + NWORK * t

            @pl.when(j < nchunks_total)
            def _():
                base = j * kchunk
                pltpu.sync_copy(sidx_hbm.at[pl.ds(base, kchunk)], sbuf)
                pltpu.sync_copy(rows, acc.at[sbuf], add=True)
        plsc.subcore_barrier()
        pltpu.sync_copy(acc.at[pl.ds(s * stripe, stripe)],
                        out_hbm.at[c, pl.ds(s * stripe, stripe)])

    return k(sidx)


def _embed_sc(tabA, xind):
    @functools.partial(
        pl.kernel, mesh=_MESH,
        out_type=jax.ShapeDtypeStruct((N, H), F32),
        scratch_types=[
            pltpu.VMEM((KS,), jnp.int32),
            pltpu.VMEM((KS, H), F32),
            pltpu.VMEM((KS, H), F32),
            pltpu.SemaphoreType.DMA,
        ],
    )
    def k(tab_hbm, xind_hbm, out_hbm, idxb, rows, accb, sem):
        c = lax.axis_index("c")
        s = lax.axis_index("s")
        w = c * 16 + s
        for t in range(4):
            j = w + NWORK * t

            @pl.when(j < N // KS)
            def _():
                base = j * KS
                _fill_const(accb, KS, _zeros16())
                for f in range(9):
                    pltpu.sync_copy(xind_hbm.at[pl.ds(f * N + base, KS)],
                                    idxb)
                    pltpu.async_copy(tab_hbm.at[idxb], rows, sem).wait()

                    def addrow(i, _):
                        for jj in range(H // 16):
                            sl = pl.ds(16 * jj, 16)
                            accb[i, sl] = accb[i, sl] + rows[i, sl]
                        return 0
                    lax.fori_loop(0, KS, addrow, 0)
                pltpu.sync_copy(accb, out_hbm.at[pl.ds(base, KS)])

    return k(tabA, xind)


def _pool_sc(h, batch):
    @functools.partial(
        pl.kernel, mesh=_MESH,
        out_type=jax.ShapeDtypeStruct((2, NG, H), F32),
        scratch_types=[
            pltpu.VMEM((KS,), jnp.int32),
            pltpu.VMEM((KS, H), F32),
            pltpu.VMEM((ZR, H), F32),
            pltpu.VMEM_SHARED((NG, H), F32),
        ],
    )
    def k(h_hbm, b_hbm, out_hbm, idxb, rows, zbuf, acc):
        c = lax.axis_index("c")
        s = lax.axis_index("s")
        w = c * 16 + s
        _fill_const(zbuf, ZR, _zeros16())
        stripe = NG // 16
        _zero_stripe(acc, zbuf, stripe, s)
        plsc.subcore_barrier()
        for t in range(4):
            j = w + NWORK * t

            @pl.when(j < N // KS)
            def _():
                base = j * KS
                pltpu.sync_copy(b_hbm.at[pl.ds(base, KS)], idxb)
                pltpu.sync_copy(h_hbm.at[pl.ds(base, KS)], rows)
                pltpu.sync_copy(rows, acc.at[idxb], add=True)
        plsc.subcore_barrier()
        pltpu.sync_copy(acc.at[pl.ds(s * stripe, stripe)],
                        out_hbm.at[c, pl.ds(s * stripe, stripe)])

    return k(h, batch)


def _cnt_tc(hist, rows):
    def body(h_ref, o_ref):
        sm = h_ref[0][:, 0:1] + h_ref[1][:, 0:1]
        o_ref[...] = jnp.broadcast_to(sm, (rows, 8))
    return pl.pallas_call(
        body,
        out_shape=jax.ShapeDtypeStruct((rows, 8), F32),
    )(hist)


def _tbuild(bond_emb):
    def body(be_ref, t_ref):
        b0 = be_ref[0]
        b1 = be_ref[1]
        b2 = be_ref[2]
        for a in range(8):
            for b in range(8):
                base = a * 64 + b * 8
                t_ref[base:base + 8, :] = b0[a:a + 1, :] + b1[b:b + 1, :] + b2
    return pl.pallas_call(
        body,
        out_shape=jax.ShapeDtypeStruct((512, H), F32),
    )(bond_emb)


def _xc_init(xclf, emb, cpad):
    def body(xcl_ref, emb_ref, o_ref):
        oh = (xcl_ref[...] ==
              lax.broadcasted_iota(jnp.int32, (cpad, 4), 1).astype(F32)
              ).astype(F32)
        o_ref[...] = jnp.dot(oh, emb_ref[...], preferred_element_type=F32)
    return pl.pallas_call(
        body,
        out_shape=jax.ShapeDtypeStruct((cpad, H), F32),
    )(xclf, emb)


_BR = 400
_NBLK = N // _BR


def _mlp1(h, aggp, epsb, w1, b1):
    def body(h_ref, agg_ref, eps_ref, w1_ref, b1_ref, y_ref, st_ref):
        i = pl.program_id(0)
        e = eps_ref[0:1, 0:1]
        z = e * h_ref[...] + agg_ref[0] + agg_ref[1]
        y = jnp.dot(z, w1_ref[...], preferred_element_type=F32) + b1_ref[...]
        y_ref[...] = y

        @pl.when(i == 0)
        def _():
            st_ref[...] = jnp.zeros_like(st_ref)
        st_ref[0:1, :] += jnp.sum(y, axis=0, keepdims=True)
        st_ref[1:2, :] += jnp.sum(y * y, axis=0, keepdims=True)

    return pl.pallas_call(
        body,
        grid=(_NBLK,),
        in_specs=[
            pl.BlockSpec((_BR, H), lambda i: (i, 0)),
            pl.BlockSpec((2, _BR, H), lambda i: (0, i, 0)),
            pl.BlockSpec((8, H), lambda i: (0, 0)),
            pl.BlockSpec((H, 2 * H), lambda i: (0, 0)),
            pl.BlockSpec((1, 2 * H), lambda i: (0, 0)),
        ],
        out_specs=[
            pl.BlockSpec((_BR, 2 * H), lambda i: (i, 0)),
            pl.BlockSpec((8, 2 * H), lambda i: (0, 0)),
        ],
        out_shape=[
            jax.ShapeDtypeStruct((N, 2 * H), F32),
            jax.ShapeDtypeStruct((8, 2 * H), F32),
        ],
    )(h, aggp, epsb, w1, b1)


def _mlp2(y1, st1, g1, be1, w2, b2):
    def body(y_ref, st_ref, g_ref, be_ref, w2_ref, b2_ref, o_ref, st2_ref):
        i = pl.program_id(0)
        mean = st_ref[0:1, :] * (1.0 / N)
        var = st_ref[1:2, :] * (1.0 / N) - mean * mean
        inv = lax.rsqrt(var + 1e-5)
        yn = (y_ref[...] - mean) * (inv * g_ref[...]) + be_ref[...]
        yn = jnp.maximum(yn, 0.0)
        o = jnp.dot(yn, w2_ref[...], preferred_element_type=F32) + b2_ref[...]
        o_ref[...] = o

        @pl.when(i == 0)
        def _():
            st2_ref[...] = jnp.zeros_like(st2_ref)
        st2_ref[0:1, :] += jnp.sum(o, axis=0, keepdims=True)
        st2_ref[1:2, :] += jnp.sum(o * o, axis=0, keepdims=True)

    return pl.pallas_call(
        body,
        grid=(_NBLK,),
        in_specs=[
            pl.BlockSpec((_BR, 2 * H), lambda i: (i, 0)),
            pl.BlockSpec((8, 2 * H), lambda i: (0, 0)),
            pl.BlockSpec((1, 2 * H), lambda i: (0, 0)),
            pl.BlockSpec((1, 2 * H), lambda i: (0, 0)),
            pl.BlockSpec((2 * H, H), lambda i: (0, 0)),
            pl.BlockSpec((1, H), lambda i: (0, 0)),
        ],
        out_specs=[
            pl.BlockSpec((_BR, H), lambda i: (i, 0)),
            pl.BlockSpec((8, H), lambda i: (0, 0)),
        ],
        out_shape=[
            jax.ShapeDtypeStruct((N, H), F32),
            jax.ShapeDtypeStruct((8, H), F32),
        ],
    )(y1, st1, g1, be1, w2, b2)


def _mlp3(y2, st2, bg, bb):
    def body(y_ref, st_ref, g_ref, b_ref, o_ref):
        mean = st_ref[0:1, :] * (1.0 / N)
        var = st_ref[1:2, :] * (1.0 / N) - mean * mean
        inv = lax.rsqrt(var + 1e-5)
        hn = (y_ref[...] - mean) * (inv * g_ref[...]) + b_ref[...]
        o_ref[...] = jnp.maximum(hn, 0.0)

    return pl.pallas_call(
        body,
        grid=(_NBLK,),
        in_specs=[
            pl.BlockSpec((_BR, H), lambda i: (i, 0)),
            pl.BlockSpec((8, H), lambda i: (0, 0)),
            pl.BlockSpec((1, H), lambda i: (0, 0)),
            pl.BlockSpec((1, H), lambda i: (0, 0)),
        ],
        out_specs=pl.BlockSpec((_BR, H), lambda i: (i, 0)),
        out_shape=jax.ShapeDtypeStruct((N, H), F32),
    )(y2, st2, bg, bb)


def _a2c_post(xc, csp, cnt, wa, ba, w1, b1, w2, b2, cpad):
    def body(xc_ref, cs_ref, cnt_ref, wa_ref, ba_ref, w1_ref, b1_ref,
             w2_ref, b2_ref, o_ref):
        sm = cs_ref[0] + cs_ref[1]
        cl = jnp.maximum(cnt_ref[:, 0:1], 1.0)
        mean = sm / cl
        t = xc_ref[...] + jnp.maximum(
            jnp.dot(mean, wa_ref[...], preferred_element_type=F32)
            + ba_ref[...], 0.0)
        u = jnp.maximum(
            jnp.dot(t, w1_ref[...], preferred_element_type=F32)
            + b1_ref[...], 0.0)
        o_ref[...] = t + jnp.dot(u, w2_ref[...], preferred_element_type=F32) \
            + b2_ref[...]

    return pl.pallas_call(
        body,
        out_shape=jax.ShapeDtypeStruct((cpad, H), F32),
    )(xc, csp, cnt, wa, ba, w1, b1, w2, b2)


def _c2a_post(h, n5, n6, c5, c6, w5, b5, w6, b6):
    def body(h_ref, n5_ref, n6_ref, c5_ref, c6_ref, w5_ref, b5_ref, w6_ref,
             b6_ref, o_ref):
        def contrib(nref, cref, wref, bref):
            sm = nref[0] + nref[1]
            cl = jnp.maximum(cref[:, 0:1], 1.0)
            mean = sm / cl
            return jnp.maximum(
                jnp.dot(mean, wref[...], preferred_element_type=F32)
                + bref[...], 0.0)
        o_ref[...] = h_ref[...] + contrib(n5_ref, c5_ref, w5_ref, b5_ref) \
            + contrib(n6_ref, c6_ref, w6_ref, b6_ref)

    return pl.pallas_call(
        body,
        grid=(_NBLK,),
        in_specs=[
            pl.BlockSpec((_BR, H), lambda i: (i, 0)),
            pl.BlockSpec((2, _BR, H), lambda i: (0, i, 0)),
            pl.BlockSpec((2, _BR, H), lambda i: (0, i, 0)),
            pl.BlockSpec((_BR, 8), lambda i: (i, 0)),
            pl.BlockSpec((_BR, 8), lambda i: (i, 0)),
            pl.BlockSpec((H, H), lambda i: (0, 0)),
            pl.BlockSpec((1, H), lambda i: (0, 0)),
            pl.BlockSpec((H, H), lambda i: (0, 0)),
            pl.BlockSpec((1, H), lambda i: (0, 0)),
        ],
        out_specs=pl.BlockSpec((_BR, H), lambda i: (i, 0)),
        out_shape=jax.ShapeDtypeStruct((N, H), F32),
    )(h, n5, n6, c5, c6, w5, b5, w6, b6)


def _final_tc(poolp, cntg, aw, ab, lw, lb):
    def body(p_ref, c_ref, aw_ref, ab_ref, lw_ref, lb_ref, o_ref):
        sm = p_ref[0] + p_ref[1]
        cl = jnp.maximum(c_ref[:, 0:1], 1.0)
        g = sm / cl
        g = jnp.maximum(
            jnp.dot(g, aw_ref[...], preferred_element_type=F32)
            + ab_ref[...], 0.0)
        o_ref[...] = jnp.dot(g, lw_ref[...], preferred_element_type=F32) \
            + lb_ref[...]

    return pl.pallas_call(
        body,
        out_shape=jax.ShapeDtypeStruct((NG, H), F32),
    )(poolp, cntg, aw, ab, lw, lb)


def kernel(x, edge_index, edge_attr, batch, x_clique_5_cycle,
           atom2clique_5_cycle, x_clique_6_cycle, atom2clique_6_cycle,
           params):
    p = params
    i32 = jnp.int32
    xcl = {'5': x_clique_5_cycle, '6': x_clique_6_cycle}
    a2c = {'5': atom2clique_5_cycle, '6': atom2clique_6_cycle}

    tabA = p['atom_emb'].reshape(9 * 64, H)
    xind = (x.astype(i32) + (jnp.arange(9, dtype=i32) * 64)[None, :]) \
        .T.reshape(-1)

    src = edge_index[0].astype(i32)
    dst = edge_index[1].astype(i32)
    ea = edge_attr.astype(i32)
    code = ea[:, 0] * 64 + ea[:, 1] * 8 + ea[:, 2]
    epadn = EPAD - E
    srcp = jnp.concatenate([src, jnp.zeros((epadn,), i32)])
    dstp = jnp.concatenate([dst, jnp.full((epadn,), N, i32)])
    codep = jnp.concatenate([code, jnp.zeros((epadn,), i32)])
    epack = jnp.stack([srcp.reshape(-1, KC), dstp.reshape(-1, KC),
                       codep.reshape(-1, KC)], axis=1)

    pidx = {}
    for kk in ('5', '6'):
        row = a2c[kk][0].astype(i32)
        col = a2c[kk][1].astype(i32)
        pr = PPAD[kk] - row.shape[0]
        a2c_g = jnp.concatenate([row, jnp.zeros((pr,), i32)])
        a2c_s = jnp.concatenate([col, jnp.full((pr,), CSZ[kk], i32)])
        c2a_g = jnp.concatenate([col, jnp.zeros((pr,), i32)])
        c2a_s = jnp.concatenate([row, jnp.full((pr,), N, i32)])
        pidx[kk] = dict(
            a2c_s=a2c_s, c2a_s=c2a_s,
            a2c_pack=jnp.stack([a2c_g.reshape(-1, KC),
                                a2c_s.reshape(-1, KC)], axis=1),
            c2a_pack=jnp.stack([c2a_g.reshape(-1, KC),
                                c2a_s.reshape(-1, KC)], axis=1),
        )

    xcf = {}
    for kk in ('5', '6'):
        xp = jnp.concatenate(
            [xcl[kk].astype(i32),
             jnp.zeros((CPAD[kk] - CSZ[kk],), i32)])
        xcf[kk] = xp.astype(F32)[:, None]

    batch_i = batch.astype(i32)

    cnt_a2c = {kk: _cnt_tc(_hist_sc(pidx[kk]['a2c_s'], CPAD[kk], KC,
                                    PPAD[kk] // KC), CPAD[kk])
               for kk in ('5', '6')}
    cnt_c2a = {kk: _cnt_tc(_hist_sc(pidx[kk]['c2a_s'], NP, KC,
                                    PPAD[kk] // KC), NP)
               for kk in ('5', '6')}
    cnt_g = _cnt_tc(_hist_sc(batch_i, NG, KS, N // KS), NG)

    h = _embed_sc(tabA, xind)
    xc = {kk: _xc_init(xcf[kk], p['cycle_emb'][kk], CPAD[kk])
          for kk in ('5', '6')}

    for li in range(len(p['layers'])):
        lp = p['layers'][li]
        nn = lp['nn']
        tbl = _tbuild(lp['bond_emb'])
        aggp = _edge_sc(h, tbl, epack)
        epsb = jnp.full((8, H), 1.0, F32) * (1.0 + lp['eps'])
        y1, st1 = _mlp1(h, aggp, epsb, nn['W1'], nn['b1'].reshape(1, -1))
        y2, st2 = _mlp2(y1, st1, nn['g1'].reshape(1, -1),
                        nn['be1'].reshape(1, -1), nn['W2'],
                        nn['b2'].reshape(1, -1))
        h = _mlp3(y2, st2, lp['bn']['g'].reshape(1, -1),
                  lp['bn']['b'].reshape(1, -1))
        csp5, csp6 = _a2c_pair_sc(h, pidx['5']['a2c_pack'],
                                  pidx['6']['a2c_pack'])
        for kk, csp in (('5', csp5), ('6', csp6)):
            cw = lp['cyc'][kk]
            xc[kk] = _a2c_post(
                xc[kk], csp, cnt_a2c[kk],
                lp['a2c'][kk]['W'], lp['a2c'][kk]['b'].reshape(1, -1),
                cw['l1']['W'], cw['l1']['b'].reshape(1, -1),
                cw['l2']['W'], cw['l2']['b'].reshape(1, -1),
                CPAD[kk])
        nsp5, nsp6 = _c2a_pair_sc(xc['5'], pidx['5']['c2a_pack'],
                                  xc['6'], pidx['6']['c2a_pack'])
        h = _c2a_post(h, nsp5, nsp6, cnt_c2a['5'], cnt_c2a['6'],
                      lp['c2a']['5']['W'], lp['c2a']['5']['b'].reshape(1, -1),
                      lp['c2a']['6']['W'], lp['c2a']['6']['b'].reshape(1, -1))

    poolp = _pool_sc(h, batch_i)
    return _final_tc(poolp, cnt_g, p['atom_lin']['W'],
                     p['atom_lin']['b'].reshape(1, -1),
                     p['lin']['W'], p['lin']['b'].reshape(1, -1))
